# baseline (speedup 1.0000x reference)
import functools

import jax
import jax.numpy as jnp
from jax import lax
from jax.experimental import pallas as pl
from jax.experimental.pallas import tpu as pltpu
from jax.experimental.pallas import tpu_sc as plsc

_NC = 2
_NS = 16
_L = 16
_NW = _NC * _NS

_B = 16384
_D = 64
_BPW = _B // _NW
_NCHUNK = 4
_CHUNK = _BPW // _NCHUNK


_mesh = plsc.VectorSubcoreMesh(core_axis_name="c", subcore_axis_name="s")


@functools.partial(
    pl.kernel,
    out_type=jax.ShapeDtypeStruct((_NW, _BPW), jnp.float32),
    mesh=_mesh,
    compiler_params=pltpu.CompilerParams(needs_layout_passes=False,
                                         use_tc_tiling_on_sc=False),
    scratch_types=[
        pltpu.VMEM((_NCHUNK, _CHUNK), jnp.int32),
        pltpu.VMEM((_NCHUNK, _CHUNK), jnp.int32),
        pltpu.VMEM((_NCHUNK, _CHUNK, _D), jnp.float32),
        pltpu.VMEM((_NCHUNK, _CHUNK, _D), jnp.float32),
        pltpu.VMEM((_BPW,), jnp.float32),
        pltpu.SemaphoreType.DMA((_NCHUNK,)),
        pltpu.SemaphoreType.DMA((_NCHUNK,)),
    ],
)
def _sc_dot_gather(idx0_hbm, idx1_hbm, f0_hbm, f1_hbm, out_hbm,
                   idx0_v, idx1_v, rows0, rows1, out_v, sem0, sem1):
    wid = lax.axis_index("s") * _NC + lax.axis_index("c")

    copies = []
    for j in range(_NCHUNK):
        pltpu.sync_copy(idx0_hbm.at[wid, j], idx0_v.at[j])
        pltpu.sync_copy(idx1_hbm.at[wid, j], idx1_v.at[j])
        copies.append(pltpu.async_copy(f0_hbm.at[idx0_v.at[j]], rows0.at[j],
                                       sem0.at[j]))
        copies.append(pltpu.async_copy(f1_hbm.at[idx1_v.at[j]], rows1.at[j],
                                       sem1.at[j]))

    last_lane = lax.iota(jnp.int32, _L) == (_L - 1)

    for j in range(_NCHUNK):
        copies[2 * j].wait()
        copies[2 * j + 1].wait()

        def body(r, carry, j=j):
            acc = (rows0[j, r, pl.ds(0, _L)] * rows1[j, r, pl.ds(0, _L)])
            for c in range(1, _D // _L):
                acc = acc + (rows0[j, r, pl.ds(c * _L, _L)]
                             * rows1[j, r, pl.ds(c * _L, _L)])
            total = plsc.cumsum(acc)
            dst = jnp.full((_L,), j * _CHUNK + r, dtype=jnp.int32)
            plsc.store_scatter(out_v, [dst], total, mask=last_lane)
            return carry

        lax.fori_loop(0, _CHUNK, body, 0)

    pltpu.sync_copy(out_v, out_hbm.at[wid])


def kernel(inputs_0, inputs_1, factor_0, factor_1):
    idx0 = inputs_0.reshape(_NW, _NCHUNK, _CHUNK)
    idx1 = inputs_1.reshape(_NW, _NCHUNK, _CHUNK)
    out = _sc_dot_gather(idx0, idx1, factor_0, factor_1)
    return out.reshape(_B)


# speedup vs baseline: 2.8185x; 2.8185x over previous
import functools

import jax
import jax.numpy as jnp
from jax import lax
from jax.experimental import pallas as pl
from jax.experimental.pallas import tpu as pltpu
from jax.experimental.pallas import tpu_sc as plsc

_NC = 2
_NS = 16
_L = 16
_NW = _NC * _NS

_B = 16384
_D = 64
_V = 1000000
_BPW = _B // _NW
_NGRP = _BPW // _L
_NSLOT = 6
_LEAD = 5

_mesh = plsc.VectorSubcoreMesh(core_axis_name="c", subcore_axis_name="s")


@functools.partial(
    pl.kernel,
    out_type=jax.ShapeDtypeStruct((_NW, _BPW), jnp.float32),
    mesh=_mesh,
    compiler_params=pltpu.CompilerParams(needs_layout_passes=False,
                                         use_tc_tiling_on_sc=True),
    scratch_types=[
        pltpu.VMEM((_NGRP, _L), jnp.int32),
        pltpu.VMEM((_NGRP, _L), jnp.int32),
        pltpu.SMEM((_BPW,), jnp.int32),
        pltpu.SMEM((_BPW,), jnp.int32),
        pltpu.VMEM((_NSLOT, _D, 128), jnp.float32),
        pltpu.VMEM((_NSLOT, _D, 128), jnp.float32),
        pltpu.VMEM((_BPW,), jnp.float32),
        pltpu.SemaphoreType.DMA((_NSLOT,)),
        pltpu.SemaphoreType.DMA((_NSLOT,)),
    ],
)
def _sc_dot_gather(idx0_hbm, idx1_hbm, f0t_hbm, f1t_hbm, out_hbm,
                   idx0_v, idx1_v, si0, si1, blk0, blk1, out_v, sem0, sem1):
    wid = lax.axis_index("s") * _NC + lax.axis_index("c")

    pltpu.sync_copy(idx0_hbm.at[wid], idx0_v)
    pltpu.sync_copy(idx1_hbm.at[wid], idx1_v)

    def fill(g, carry):
        v0 = idx0_v[g, :]
        v1 = idx1_v[g, :]
        for j in range(_L):
            si0[g * _L + j] = v0[j]
            si1[g * _L + j] = v1[j]
        return carry

    lax.fori_loop(0, _NGRP, fill, 0)

    def fire(k, slot):
        t0 = pl.multiple_of((si0[k] // 128) * 128, 128)
        t1 = pl.multiple_of((si1[k] // 128) * 128, 128)
        pltpu.async_copy(f0t_hbm.at[:, pl.ds(t0, 128)], blk0.at[slot],
                         sem0.at[slot])
        pltpu.async_copy(f1t_hbm.at[:, pl.ds(t1, 128)], blk1.at[slot],
                         sem1.at[slot])

    for k in range(_LEAD):
        fire(k, k)

    lanes = lax.iota(jnp.int32, _L)
    last_lane = lanes == (_L - 1)

    def body(i, carry):
        @pl.when(i < _BPW - _LEAD)
        def _():
            fire(i + _LEAD, (i + _LEAD) % _NSLOT)

        slot = i % _NSLOT
        pltpu.make_async_copy(f0t_hbm.at[:, pl.ds(0, 128)], blk0.at[slot],
                              sem0.at[slot]).wait()
        pltpu.make_async_copy(f1t_hbm.at[:, pl.ds(0, 128)], blk1.at[slot],
                              sem1.at[slot]).wait()

        c0 = jnp.full((_L,), si0[i] % 128, jnp.int32)
        c1 = jnp.full((_L,), si1[i] % 128, jnp.int32)
        slot_spl = jnp.full((_L,), slot, jnp.int32)
        acc = None
        for q in range(_D // _L):
            rows = lanes + (q * _L)
            e0 = plsc.load_gather(blk0, [slot_spl, rows, c0])
            e1 = plsc.load_gather(blk1, [slot_spl, rows, c1])
            p = e0 * e1
            acc = p if acc is None else acc + p
        plsc.store_scatter(out_v, [jnp.full((_L,), i, jnp.int32)],
                           plsc.cumsum(acc), mask=last_lane)
        return carry

    lax.fori_loop(0, _BPW, body, 0)

    pltpu.sync_copy(out_v, out_hbm.at[wid])


def kernel(inputs_0, inputs_1, factor_0, factor_1):
    idx0 = inputs_0.reshape(_NW, _NGRP, _L)
    idx1 = inputs_1.reshape(_NW, _NGRP, _L)
    out = _sc_dot_gather(idx0, idx1, factor_0.T, factor_1.T)
    return out.reshape(_B)


# speedup vs baseline: 2.8267x; 1.0029x over previous
import functools

import jax
import jax.numpy as jnp
from jax import lax
from jax.experimental import pallas as pl
from jax.experimental.pallas import tpu as pltpu
from jax.experimental.pallas import tpu_sc as plsc

_NC = 2
_NS = 16
_L = 16
_NW = _NC * _NS

_B = 16384
_D = 64
_V = 1000000
_BPW = _B // _NW
_NGRP = _BPW // _L
_NSLOT = 7
_LEAD = 6

_mesh = plsc.VectorSubcoreMesh(core_axis_name="c", subcore_axis_name="s")


@functools.partial(
    pl.kernel,
    out_type=jax.ShapeDtypeStruct((_NW, _BPW), jnp.float32),
    mesh=_mesh,
    compiler_params=pltpu.CompilerParams(needs_layout_passes=False,
                                         use_tc_tiling_on_sc=True),
    scratch_types=[
        pltpu.VMEM((_NGRP, _L), jnp.int32),
        pltpu.VMEM((_NGRP, _L), jnp.int32),
        pltpu.SMEM((_BPW,), jnp.int32),
        pltpu.SMEM((_BPW,), jnp.int32),
        pltpu.VMEM((_NSLOT, _D, 128), jnp.float32),
        pltpu.VMEM((_NSLOT, _D, 128), jnp.float32),
        pltpu.VMEM((_BPW,), jnp.float32),
        pltpu.SemaphoreType.DMA((_NSLOT,)),
        pltpu.SemaphoreType.DMA((_NSLOT,)),
    ],
)
def _sc_dot_gather(idx0_hbm, idx1_hbm, f0t_hbm, f1t_hbm, out_hbm,
                   idx0_v, idx1_v, si0, si1, blk0, blk1, out_v, sem0, sem1):
    wid = lax.axis_index("s") * _NC + lax.axis_index("c")

    pltpu.sync_copy(idx0_hbm.at[wid], idx0_v)
    pltpu.sync_copy(idx1_hbm.at[wid], idx1_v)

    def fill(g, carry):
        v0 = idx0_v[g, :]
        v1 = idx1_v[g, :]
        for j in range(_L):
            si0[g * _L + j] = v0[j]
            si1[g * _L + j] = v1[j]
        return carry

    lax.fori_loop(0, _NGRP, fill, 0)

    def fire(k, slot):
        t0 = pl.multiple_of((si0[k] // 128) * 128, 128)
        t1 = pl.multiple_of((si1[k] // 128) * 128, 128)
        pltpu.async_copy(f0t_hbm.at[:, pl.ds(t0, 128)], blk0.at[slot],
                         sem0.at[slot])
        pltpu.async_copy(f1t_hbm.at[:, pl.ds(t1, 128)], blk1.at[slot],
                         sem1.at[slot])

    for k in range(_LEAD):
        fire(k, k)

    lanes = lax.iota(jnp.int32, _L)
    last_lane = lanes == (_L - 1)

    def body(i, carry):
        @pl.when(i < _BPW - _LEAD)
        def _():
            fire(i + _LEAD, (i + _LEAD) % _NSLOT)

        slot = i % _NSLOT
        pltpu.make_async_copy(f0t_hbm.at[:, pl.ds(0, 128)], blk0.at[slot],
                              sem0.at[slot]).wait()
        pltpu.make_async_copy(f1t_hbm.at[:, pl.ds(0, 128)], blk1.at[slot],
                              sem1.at[slot]).wait()

        c0 = jnp.full((_L,), si0[i] % 128, jnp.int32)
        c1 = jnp.full((_L,), si1[i] % 128, jnp.int32)
        slot_spl = jnp.full((_L,), slot, jnp.int32)
        acc = None
        for q in range(_D // _L):
            rows = lanes + (q * _L)
            e0 = plsc.load_gather(blk0, [slot_spl, rows, c0])
            e1 = plsc.load_gather(blk1, [slot_spl, rows, c1])
            p = e0 * e1
            acc = p if acc is None else acc + p
        plsc.store_scatter(out_v, [jnp.full((_L,), i, jnp.int32)],
                           plsc.cumsum(acc), mask=last_lane)
        return carry

    lax.fori_loop(0, _BPW, body, 0)

    pltpu.sync_copy(out_v, out_hbm.at[wid])


def kernel(inputs_0, inputs_1, factor_0, factor_1):
    idx0 = inputs_0.reshape(_NW, _NGRP, _L)
    idx1 = inputs_1.reshape(_NW, _NGRP, _L)
    out = _sc_dot_gather(idx0, idx1, factor_0.T, factor_1.T)
    return out.reshape(_B)
